# baseline (speedup 1.0000x reference)
import jax
import jax.numpy as jnp
from jax import lax
from jax.experimental import pallas as pl
from jax.experimental.pallas import tpu as pltpu
from jax.experimental.pallas import tpu_sc as plsc

N_NODES = 10000
D = 256
H = 128
E = 160000
NT = 16
K = 48
TPC = 212
EPAD = NT * TPC * K
ACC_ROWS = 10112
STRIPE = ACC_ROWS // NT
TAIL = N_NODES - 15 * STRIPE
RBLK = 2000
GRID = N_NODES // RBLK
NBUF = 9
RB = 7
GA = 5
IA = 7
SL = 2


def _agg_body(h_lo, h_hi, idx4, zeros_hbm, out_lo, out_hi,
              idx_v, rows_v, acc, isem, gsem, ssem):
    c = lax.axis_index("c")
    s = lax.axis_index("s")

    pltpu.sync_copy(zeros_hbm.at[pl.ds(s * STRIPE, STRIPE)],
                    acc.at[pl.ds(s * STRIPE, STRIPE)])
    plsc.subcore_barrier()

    def idx_load(j, slot):
        pltpu.async_copy(idx4.at[s].at[j], idx_v.at[slot], isem.at[slot])

    def idx_wait(j, slot):
        pltpu.make_async_copy(idx4.at[s].at[j], idx_v.at[slot],
                              isem.at[slot]).wait()

    def run(h_ref):
        for j in range(IA):
            idx_load(j, j)
        for j in range(GA):
            idx_wait(j, j)
            pltpu.async_copy(h_ref.at[idx_v.at[j].at[0]], rows_v.at[j],
                             gsem.at[j])

        def chunk(j, carry):
            bi = lax.rem(j, NBUF)
            br = lax.rem(j, RB)
            bid = lax.rem(j + IA, NBUF)
            big = lax.rem(j + GA, NBUF)
            brg = lax.rem(j + GA, RB)
            pltpu.make_async_copy(h_ref.at[idx_v.at[bi].at[0]], rows_v.at[br],
                                  gsem.at[br]).wait()
            pltpu.async_copy(rows_v.at[br], acc.at[idx_v.at[bi].at[1]],
                             ssem.at[br], add=True)

            @pl.when(j >= SL)
            def _():
                pltpu.make_async_copy(rows_v.at[brg], acc.at[idx_v.at[bid].at[1]],
                                      ssem.at[brg]).wait()

            @pl.when(j + IA < TPC)
            def _():
                idx_load(j + IA, bid)

            @pl.when(j + GA < TPC)
            def _():
                idx_wait(j + GA, big)
                pltpu.async_copy(h_ref.at[idx_v.at[big].at[0]], rows_v.at[brg],
                                 gsem.at[brg])

            return carry
        lax.fori_loop(0, TPC, chunk, 0, unroll=4)

        for j in range(TPC - SL, TPC):
            pltpu.make_async_copy(rows_v.at[j % RB],
                                  acc.at[idx_v.at[j % NBUF].at[1]],
                                  ssem.at[j % RB]).wait()

    @pl.when(c == 0)
    def _():
        run(h_lo)

    @pl.when(c == 1)
    def _():
        run(h_hi)

    plsc.subcore_barrier()

    def writeback(out_ref):
        @pl.when(s < NT - 1)
        def _():
            pltpu.sync_copy(acc.at[pl.ds(s * STRIPE, STRIPE)],
                            out_ref.at[pl.ds(s * STRIPE, STRIPE)])

        @pl.when(s == NT - 1)
        def _():
            pltpu.sync_copy(acc.at[pl.ds((NT - 1) * STRIPE, TAIL)],
                            out_ref.at[pl.ds((NT - 1) * STRIPE, TAIL)])

    @pl.when(c == 0)
    def _():
        writeback(out_lo)

    @pl.when(c == 1)
    def _():
        writeback(out_hi)


_agg_cache = []


def _agg(*args):
    if not _agg_cache:
        _agg_cache.append(pl.kernel(
            _agg_body,
            out_type=(jax.ShapeDtypeStruct((N_NODES, H), jnp.float32),
                      jax.ShapeDtypeStruct((N_NODES, H), jnp.float32)),
            mesh=plsc.VectorSubcoreMesh(core_axis_name="c", subcore_axis_name="s",
                                        num_cores=2, num_subcores=NT),
            scratch_types=[
                pltpu.VMEM((NBUF, 2, K), jnp.int32),
                pltpu.VMEM((RB, K, H), jnp.float32),
                pltpu.VMEM_SHARED((ACC_ROWS, H), jnp.float32),
                pltpu.SemaphoreType.DMA((NBUF,)),
                pltpu.SemaphoreType.DMA((RB,)),
                pltpu.SemaphoreType.DMA((RB,)),
            ],
        ))
    return _agg_cache[0](*args)


def _embed_body(x_ref, wsp_ref, bsp_ref, tt_ref, pt_ref, pp_ref, wpp_ref,
                bpp_ref, hs_lo, hs_hi, hr_lo, hr_hi):
    feat = jnp.log1p(x_ref[...])
    hs = jnp.tanh(feat * wsp_ref[...] + bsp_ref[...])
    hs_lo[...] = hs[:, :H]
    hs_hi[...] = hs[:, H:]

    ids = pt_ref[...]
    oh = (ids == lax.broadcasted_iota(jnp.int32, (ids.shape[0], 8), 1))
    emb = jnp.dot(oh.astype(jnp.float32), tt_ref[...],
                  preferred_element_type=jnp.float32)
    pp = pp_ref[...]
    lin = jnp.zeros_like(emb)
    for kk in range(4):
        lin = lin + pp[:, kk:kk + 1] * wpp_ref[kk:kk + 1, :]
    hr = jnp.tanh(emb + lin + bpp_ref[...])
    hr_lo[...] = hr[:, :H]
    hr_hi[...] = hr[:, H:]


_embed = pl.pallas_call(
    _embed_body,
    grid=(GRID,),
    in_specs=[
        pl.BlockSpec((RBLK, 1), lambda i: (i, 0)),
        pl.BlockSpec((1, D), lambda i: (0, 0)),
        pl.BlockSpec((1, D), lambda i: (0, 0)),
        pl.BlockSpec((8, D), lambda i: (0, 0)),
        pl.BlockSpec((RBLK, 1), lambda i: (i, 0)),
        pl.BlockSpec((RBLK, 4), lambda i: (i, 0)),
        pl.BlockSpec((4, D), lambda i: (0, 0)),
        pl.BlockSpec((1, D), lambda i: (0, 0)),
    ],
    out_specs=[
        pl.BlockSpec((RBLK, H), lambda i: (i, 0)),
        pl.BlockSpec((RBLK, H), lambda i: (i, 0)),
        pl.BlockSpec((RBLK, H), lambda i: (i, 0)),
        pl.BlockSpec((RBLK, H), lambda i: (i, 0)),
    ],
    out_shape=[jax.ShapeDtypeStruct((N_NODES, H), jnp.float32)] * 4,
)


def _update_body(hlo_ref, hhi_ref, alo_ref, ahi_ref, w_ref, b_ref,
                 ylo_ref, yhi_ref, ysum_ref):
    i = pl.program_id(0)
    m = (jnp.dot(alo_ref[...], w_ref[:H, :], preferred_element_type=jnp.float32)
         + jnp.dot(ahi_ref[...], w_ref[H:, :], preferred_element_type=jnp.float32))
    h = jnp.concatenate([hlo_ref[...], hhi_ref[...]], axis=1)
    y = jnp.maximum(h + m + b_ref[...], 0.0)
    ylo_ref[...] = y[:, :H]
    yhi_ref[...] = y[:, H:]

    @pl.when(i == 0)
    def _():
        ysum_ref[...] = jnp.zeros_like(ysum_ref)

    ysum_ref[...] += jnp.sum(y, axis=0, keepdims=True)


_update = pl.pallas_call(
    _update_body,
    grid=(GRID,),
    in_specs=[
        pl.BlockSpec((RBLK, H), lambda i: (i, 0)),
        pl.BlockSpec((RBLK, H), lambda i: (i, 0)),
        pl.BlockSpec((RBLK, H), lambda i: (i, 0)),
        pl.BlockSpec((RBLK, H), lambda i: (i, 0)),
        pl.BlockSpec((D, D), lambda i: (0, 0)),
        pl.BlockSpec((1, D), lambda i: (0, 0)),
    ],
    out_specs=[
        pl.BlockSpec((RBLK, H), lambda i: (i, 0)),
        pl.BlockSpec((RBLK, H), lambda i: (i, 0)),
        pl.BlockSpec((1, D), lambda i: (0, 0)),
    ],
    out_shape=[
        jax.ShapeDtypeStruct((N_NODES, H), jnp.float32),
        jax.ShapeDtypeStruct((N_NODES, H), jnp.float32),
        jax.ShapeDtypeStruct((1, D), jnp.float32),
    ],
)


def kernel(initial_state, edge_species, edge_reactions, propensity_type_ids,
           propensity_params, W_sp, b_sp, type_table, W_pp, b_pp,
           Ws2r, Wr2s, br, bs):
    n_layers = Ws2r.shape[0]

    es = edge_species.astype(jnp.int32)
    er = edge_reactions.astype(jnp.int32)
    pad = EPAD - E
    es_src = jnp.concatenate([es, jnp.zeros((pad,), jnp.int32)]).reshape(NT, TPC, 1, K)
    er_src = jnp.concatenate([er, jnp.zeros((pad,), jnp.int32)]).reshape(NT, TPC, 1, K)
    es_dst = jnp.concatenate([es, jnp.full((pad,), N_NODES, jnp.int32)]).reshape(NT, TPC, 1, K)
    er_dst = jnp.concatenate([er, jnp.full((pad,), N_NODES, jnp.int32)]).reshape(NT, TPC, 1, K)
    s2r_idx = jnp.concatenate([es_src, er_dst], axis=2)
    r2s_idx = jnp.concatenate([er_src, es_dst], axis=2)
    zeros = jnp.zeros((ACC_ROWS, H), jnp.float32)

    hs_lo, hs_hi, hr_lo, hr_hi = _embed(
        initial_state.reshape(N_NODES, 1), W_sp, b_sp.reshape(1, D), type_table,
        propensity_type_ids.astype(jnp.int32).reshape(N_NODES, 1),
        propensity_params, W_pp, b_pp.reshape(1, D))

    sum_r = sum_s = None
    for l in range(n_layers):
        a_lo, a_hi = _agg(hs_lo, hs_hi, s2r_idx, zeros)
        hr_lo, hr_hi, sum_r = _update(hr_lo, hr_hi, a_lo, a_hi,
                                      Ws2r[l], br[l].reshape(1, D))
        a_lo, a_hi = _agg(hr_lo, hr_hi, r2s_idx, zeros)
        hs_lo, hs_hi, sum_s = _update(hs_lo, hs_hi, a_lo, a_hi,
                                      Wr2s[l], bs[l].reshape(1, D))

    h_s = jnp.concatenate([hs_lo, hs_hi], axis=1)
    h_r = jnp.concatenate([hr_lo, hr_hi], axis=1)
    context = jnp.concatenate([sum_s[0], sum_r[0]]) * (1.0 / N_NODES)
    return h_s, h_r, context


# speedup vs baseline: 1.0024x; 1.0024x over previous
import jax
import jax.numpy as jnp
from jax import lax
from jax.experimental import pallas as pl
from jax.experimental.pallas import tpu as pltpu
from jax.experimental.pallas import tpu_sc as plsc

N_NODES = 10000
D = 256
H = 128
E = 160000
NT = 16
K = 48
TPC = 212
EPAD = NT * TPC * K
ACC_ROWS = 10112
STRIPE = ACC_ROWS // NT
TAIL = N_NODES - 15 * STRIPE
RBLK = 2000
GRID = N_NODES // RBLK
NBUF = 9
RB = 7
GA = 5
IA = 7
SL = 2


def _agg_body(h_lo, h_hi, idx4, zeros_hbm, out_lo, out_hi,
              idx_v, rows_v, acc, isem, gsem, ssem):
    c = lax.axis_index("c")
    s = lax.axis_index("s")

    pltpu.sync_copy(zeros_hbm.at[pl.ds(s * STRIPE, STRIPE)],
                    acc.at[pl.ds(s * STRIPE, STRIPE)])
    plsc.subcore_barrier()

    def idx_load(j, slot):
        pltpu.async_copy(idx4.at[s].at[j], idx_v.at[slot], isem.at[slot])

    def idx_wait(j, slot):
        pltpu.make_async_copy(idx4.at[s].at[j], idx_v.at[slot],
                              isem.at[slot]).wait()

    def run(h_ref):
        for j in range(IA):
            idx_load(j, j)
        for j in range(GA):
            idx_wait(j, j)
            pltpu.async_copy(h_ref.at[idx_v.at[j].at[0]], rows_v.at[j],
                             gsem.at[j])

        def chunk(j, carry):
            bi = lax.rem(j, NBUF)
            br = lax.rem(j, RB)
            bid = lax.rem(j + IA, NBUF)
            big = lax.rem(j + GA, NBUF)
            brg = lax.rem(j + GA, RB)
            pltpu.make_async_copy(h_ref.at[idx_v.at[bi].at[0]], rows_v.at[br],
                                  gsem.at[br]).wait()
            pltpu.async_copy(rows_v.at[br], acc.at[idx_v.at[bi].at[1]],
                             ssem.at[br], add=True)

            @pl.when(j >= SL)
            def _():
                pltpu.make_async_copy(rows_v.at[brg], acc.at[idx_v.at[bid].at[1]],
                                      ssem.at[brg]).wait()

            @pl.when(j + IA < TPC)
            def _():
                idx_load(j + IA, bid)

            @pl.when(j + GA < TPC)
            def _():
                idx_wait(j + GA, big)
                pltpu.async_copy(h_ref.at[idx_v.at[big].at[0]], rows_v.at[brg],
                                 gsem.at[brg])

            return carry
        lax.fori_loop(0, TPC, chunk, 0)

        for j in range(TPC - SL, TPC):
            pltpu.make_async_copy(rows_v.at[j % RB],
                                  acc.at[idx_v.at[j % NBUF].at[1]],
                                  ssem.at[j % RB]).wait()

    @pl.when(c == 0)
    def _():
        run(h_lo)

    @pl.when(c == 1)
    def _():
        run(h_hi)

    plsc.subcore_barrier()

    def writeback(out_ref):
        @pl.when(s < NT - 1)
        def _():
            pltpu.sync_copy(acc.at[pl.ds(s * STRIPE, STRIPE)],
                            out_ref.at[pl.ds(s * STRIPE, STRIPE)])

        @pl.when(s == NT - 1)
        def _():
            pltpu.sync_copy(acc.at[pl.ds((NT - 1) * STRIPE, TAIL)],
                            out_ref.at[pl.ds((NT - 1) * STRIPE, TAIL)])

    @pl.when(c == 0)
    def _():
        writeback(out_lo)

    @pl.when(c == 1)
    def _():
        writeback(out_hi)


_agg_cache = []


def _agg(*args):
    if not _agg_cache:
        _agg_cache.append(pl.kernel(
            _agg_body,
            out_type=(jax.ShapeDtypeStruct((N_NODES, H), jnp.float32),
                      jax.ShapeDtypeStruct((N_NODES, H), jnp.float32)),
            mesh=plsc.VectorSubcoreMesh(core_axis_name="c", subcore_axis_name="s",
                                        num_cores=2, num_subcores=NT),
            scratch_types=[
                pltpu.VMEM((NBUF, 2, K), jnp.int32),
                pltpu.VMEM((RB, K, H), jnp.float32),
                pltpu.VMEM_SHARED((ACC_ROWS, H), jnp.float32),
                pltpu.SemaphoreType.DMA((NBUF,)),
                pltpu.SemaphoreType.DMA((RB,)),
                pltpu.SemaphoreType.DMA((RB,)),
            ],
        ))
    return _agg_cache[0](*args)


def _embed_body(x_ref, wsp_ref, bsp_ref, tt_ref, pt_ref, pp_ref, wpp_ref,
                bpp_ref, hs_lo, hs_hi, hr_lo, hr_hi):
    feat = jnp.log1p(x_ref[...])
    hs = jnp.tanh(feat * wsp_ref[...] + bsp_ref[...])
    hs_lo[...] = hs[:, :H]
    hs_hi[...] = hs[:, H:]

    ids = pt_ref[...]
    oh = (ids == lax.broadcasted_iota(jnp.int32, (ids.shape[0], 8), 1))
    emb = jnp.dot(oh.astype(jnp.float32), tt_ref[...],
                  preferred_element_type=jnp.float32)
    pp = pp_ref[...]
    lin = jnp.zeros_like(emb)
    for kk in range(4):
        lin = lin + pp[:, kk:kk + 1] * wpp_ref[kk:kk + 1, :]
    hr = jnp.tanh(emb + lin + bpp_ref[...])
    hr_lo[...] = hr[:, :H]
    hr_hi[...] = hr[:, H:]


_embed = pl.pallas_call(
    _embed_body,
    grid=(GRID,),
    in_specs=[
        pl.BlockSpec((RBLK, 1), lambda i: (i, 0)),
        pl.BlockSpec((1, D), lambda i: (0, 0)),
        pl.BlockSpec((1, D), lambda i: (0, 0)),
        pl.BlockSpec((8, D), lambda i: (0, 0)),
        pl.BlockSpec((RBLK, 1), lambda i: (i, 0)),
        pl.BlockSpec((RBLK, 4), lambda i: (i, 0)),
        pl.BlockSpec((4, D), lambda i: (0, 0)),
        pl.BlockSpec((1, D), lambda i: (0, 0)),
    ],
    out_specs=[
        pl.BlockSpec((RBLK, H), lambda i: (i, 0)),
        pl.BlockSpec((RBLK, H), lambda i: (i, 0)),
        pl.BlockSpec((RBLK, H), lambda i: (i, 0)),
        pl.BlockSpec((RBLK, H), lambda i: (i, 0)),
    ],
    out_shape=[jax.ShapeDtypeStruct((N_NODES, H), jnp.float32)] * 4,
)


def _update_body(hlo_ref, hhi_ref, alo_ref, ahi_ref, w_ref, b_ref,
                 ylo_ref, yhi_ref, ysum_ref):
    i = pl.program_id(0)
    m = (jnp.dot(alo_ref[...], w_ref[:H, :], preferred_element_type=jnp.float32)
         + jnp.dot(ahi_ref[...], w_ref[H:, :], preferred_element_type=jnp.float32))
    h = jnp.concatenate([hlo_ref[...], hhi_ref[...]], axis=1)
    y = jnp.maximum(h + m + b_ref[...], 0.0)
    ylo_ref[...] = y[:, :H]
    yhi_ref[...] = y[:, H:]

    @pl.when(i == 0)
    def _():
        ysum_ref[...] = jnp.zeros_like(ysum_ref)

    ysum_ref[...] += jnp.sum(y, axis=0, keepdims=True)


_update = pl.pallas_call(
    _update_body,
    grid=(GRID,),
    in_specs=[
        pl.BlockSpec((RBLK, H), lambda i: (i, 0)),
        pl.BlockSpec((RBLK, H), lambda i: (i, 0)),
        pl.BlockSpec((RBLK, H), lambda i: (i, 0)),
        pl.BlockSpec((RBLK, H), lambda i: (i, 0)),
        pl.BlockSpec((D, D), lambda i: (0, 0)),
        pl.BlockSpec((1, D), lambda i: (0, 0)),
    ],
    out_specs=[
        pl.BlockSpec((RBLK, H), lambda i: (i, 0)),
        pl.BlockSpec((RBLK, H), lambda i: (i, 0)),
        pl.BlockSpec((1, D), lambda i: (0, 0)),
    ],
    out_shape=[
        jax.ShapeDtypeStruct((N_NODES, H), jnp.float32),
        jax.ShapeDtypeStruct((N_NODES, H), jnp.float32),
        jax.ShapeDtypeStruct((1, D), jnp.float32),
    ],
)


def kernel(initial_state, edge_species, edge_reactions, propensity_type_ids,
           propensity_params, W_sp, b_sp, type_table, W_pp, b_pp,
           Ws2r, Wr2s, br, bs):
    n_layers = Ws2r.shape[0]

    es = edge_species.astype(jnp.int32)
    er = edge_reactions.astype(jnp.int32)
    pad = EPAD - E
    es_src = jnp.concatenate([es, jnp.zeros((pad,), jnp.int32)]).reshape(NT, TPC, 1, K)
    er_src = jnp.concatenate([er, jnp.zeros((pad,), jnp.int32)]).reshape(NT, TPC, 1, K)
    es_dst = jnp.concatenate([es, jnp.full((pad,), N_NODES, jnp.int32)]).reshape(NT, TPC, 1, K)
    er_dst = jnp.concatenate([er, jnp.full((pad,), N_NODES, jnp.int32)]).reshape(NT, TPC, 1, K)
    s2r_idx = jnp.concatenate([es_src, er_dst], axis=2)
    r2s_idx = jnp.concatenate([er_src, es_dst], axis=2)
    zeros = jnp.zeros((ACC_ROWS, H), jnp.float32)

    hs_lo, hs_hi, hr_lo, hr_hi = _embed(
        initial_state.reshape(N_NODES, 1), W_sp, b_sp.reshape(1, D), type_table,
        propensity_type_ids.astype(jnp.int32).reshape(N_NODES, 1),
        propensity_params, W_pp, b_pp.reshape(1, D))

    sum_r = sum_s = None
    for l in range(n_layers):
        a_lo, a_hi = _agg(hs_lo, hs_hi, s2r_idx, zeros)
        hr_lo, hr_hi, sum_r = _update(hr_lo, hr_hi, a_lo, a_hi,
                                      Ws2r[l], br[l].reshape(1, D))
        a_lo, a_hi = _agg(hr_lo, hr_hi, r2s_idx, zeros)
        hs_lo, hs_hi, sum_s = _update(hs_lo, hs_hi, a_lo, a_hi,
                                      Wr2s[l], bs[l].reshape(1, D))

    h_s = jnp.concatenate([hs_lo, hs_hi], axis=1)
    h_r = jnp.concatenate([hr_lo, hr_hi], axis=1)
    context = jnp.concatenate([sum_s[0], sum_r[0]]) * (1.0 / N_NODES)
    return h_s, h_r, context


# speedup vs baseline: 1.7733x; 1.7691x over previous
import jax
import jax.numpy as jnp
from jax import lax
from jax.experimental import pallas as pl
from jax.experimental.pallas import tpu as pltpu
from jax.experimental.pallas import tpu_sc as plsc

N_NODES = 10000
D = 256
H = 128
E = 160000
NT = 16
K = 48
TPC = 209
EPAD = NT * TPC * K
ACC_ROWS = 10112
STRIPE = ACC_ROWS // NT
TAIL = N_NODES - 15 * STRIPE
RBLK = 2000
GRID = N_NODES // RBLK
NBUF = 9
RB = 7
GA = 5
IA = 7
SL = 2


def _agg_body(h_lo, h_hi, src3, dst3, zeros_hbm, out_lo, out_hi,
              src_v, dst_v, rows_v, acc, isem, gsem, ssem):
    c = lax.axis_index("c")
    s = lax.axis_index("s")

    pltpu.sync_copy(zeros_hbm.at[pl.ds(s * STRIPE, STRIPE)],
                    acc.at[pl.ds(s * STRIPE, STRIPE)])
    plsc.subcore_barrier()

    def idx_load(j, slot):
        pltpu.async_copy(src3.at[s].at[j], src_v.at[slot], isem.at[slot])
        pltpu.async_copy(dst3.at[s].at[j], dst_v.at[slot], isem.at[slot])

    def idx_wait(j, slot):
        pltpu.make_async_copy(src3.at[s].at[j], src_v.at[slot],
                              isem.at[slot]).wait()
        pltpu.make_async_copy(dst3.at[s].at[j], dst_v.at[slot],
                              isem.at[slot]).wait()

    def run(h_ref):
        for j in range(IA):
            idx_load(j, j)
        for j in range(GA):
            idx_wait(j, j)
            pltpu.async_copy(h_ref.at[src_v.at[j]], rows_v.at[j], gsem.at[j])

        def chunk(j, carry):
            bi = lax.rem(j, NBUF)
            br = lax.rem(j, RB)
            bid = lax.rem(j + IA, NBUF)
            big = lax.rem(j + GA, NBUF)
            brg = lax.rem(j + GA, RB)
            pltpu.make_async_copy(h_ref.at[src_v.at[bi]], rows_v.at[br],
                                  gsem.at[br]).wait()
            pltpu.async_copy(rows_v.at[br], acc.at[dst_v.at[bi]], ssem.at[br],
                             add=True)

            @pl.when(j >= SL)
            def _():
                pltpu.make_async_copy(rows_v.at[brg], acc.at[dst_v.at[bid]],
                                      ssem.at[brg]).wait()

            @pl.when(j + IA < TPC)
            def _():
                idx_load(j + IA, bid)

            @pl.when(j + GA < TPC)
            def _():
                idx_wait(j + GA, big)
                pltpu.async_copy(h_ref.at[src_v.at[big]], rows_v.at[brg],
                                 gsem.at[brg])

            return carry
        lax.fori_loop(0, TPC, chunk, 0)

        for j in range(TPC - SL, TPC):
            pltpu.make_async_copy(rows_v.at[j % RB], acc.at[dst_v.at[j % NBUF]],
                                  ssem.at[j % RB]).wait()

    @pl.when(c == 0)
    def _():
        run(h_lo)

    @pl.when(c == 1)
    def _():
        run(h_hi)

    plsc.subcore_barrier()

    def writeback(out_ref):
        @pl.when(s < NT - 1)
        def _():
            pltpu.sync_copy(acc.at[pl.ds(s * STRIPE, STRIPE)],
                            out_ref.at[pl.ds(s * STRIPE, STRIPE)])

        @pl.when(s == NT - 1)
        def _():
            pltpu.sync_copy(acc.at[pl.ds((NT - 1) * STRIPE, TAIL)],
                            out_ref.at[pl.ds((NT - 1) * STRIPE, TAIL)])

    @pl.when(c == 0)
    def _():
        writeback(out_lo)

    @pl.when(c == 1)
    def _():
        writeback(out_hi)


_agg_cache = []


def _agg(*args):
    if not _agg_cache:
        _agg_cache.append(pl.kernel(
            _agg_body,
            out_type=(jax.ShapeDtypeStruct((N_NODES, H), jnp.float32),
                      jax.ShapeDtypeStruct((N_NODES, H), jnp.float32)),
            mesh=plsc.VectorSubcoreMesh(core_axis_name="c", subcore_axis_name="s",
                                        num_cores=2, num_subcores=NT),
            scratch_types=[
                pltpu.VMEM((NBUF, K), jnp.int32),
                pltpu.VMEM((NBUF, K), jnp.int32),
                pltpu.VMEM((RB, K, H), jnp.float32),
                pltpu.VMEM_SHARED((ACC_ROWS, H), jnp.float32),
                pltpu.SemaphoreType.DMA((NBUF,)),
                pltpu.SemaphoreType.DMA((RB,)),
                pltpu.SemaphoreType.DMA((RB,)),
            ],
        ))
    return _agg_cache[0](*args)


def _embed_body(x_ref, wsp_ref, bsp_ref, tt_ref, pt_ref, pp_ref, wpp_ref,
                bpp_ref, hs_lo, hs_hi, hr_lo, hr_hi):
    feat = jnp.log1p(x_ref[...])
    hs = jnp.tanh(feat * wsp_ref[...] + bsp_ref[...])
    hs_lo[...] = hs[:, :H]
    hs_hi[...] = hs[:, H:]

    ids = pt_ref[...]
    oh = (ids == lax.broadcasted_iota(jnp.int32, (ids.shape[0], 8), 1))
    emb = jnp.dot(oh.astype(jnp.float32), tt_ref[...],
                  preferred_element_type=jnp.float32)
    pp = pp_ref[...]
    lin = jnp.zeros_like(emb)
    for kk in range(4):
        lin = lin + pp[:, kk:kk + 1] * wpp_ref[kk:kk + 1, :]
    hr = jnp.tanh(emb + lin + bpp_ref[...])
    hr_lo[...] = hr[:, :H]
    hr_hi[...] = hr[:, H:]


_embed = pl.pallas_call(
    _embed_body,
    grid=(GRID,),
    in_specs=[
        pl.BlockSpec((RBLK, 1), lambda i: (i, 0)),
        pl.BlockSpec((1, D), lambda i: (0, 0)),
        pl.BlockSpec((1, D), lambda i: (0, 0)),
        pl.BlockSpec((8, D), lambda i: (0, 0)),
        pl.BlockSpec((RBLK, 1), lambda i: (i, 0)),
        pl.BlockSpec((RBLK, 4), lambda i: (i, 0)),
        pl.BlockSpec((4, D), lambda i: (0, 0)),
        pl.BlockSpec((1, D), lambda i: (0, 0)),
    ],
    out_specs=[
        pl.BlockSpec((RBLK, H), lambda i: (i, 0)),
        pl.BlockSpec((RBLK, H), lambda i: (i, 0)),
        pl.BlockSpec((RBLK, H), lambda i: (i, 0)),
        pl.BlockSpec((RBLK, H), lambda i: (i, 0)),
    ],
    out_shape=[jax.ShapeDtypeStruct((N_NODES, H), jnp.float32)] * 4,
)


def _update_body(hlo_ref, hhi_ref, alo_ref, ahi_ref, w_ref, b_ref,
                 ylo_ref, yhi_ref, ysum_ref):
    i = pl.program_id(0)
    m = (jnp.dot(alo_ref[...], w_ref[:H, :], preferred_element_type=jnp.float32)
         + jnp.dot(ahi_ref[...], w_ref[H:, :], preferred_element_type=jnp.float32))
    h = jnp.concatenate([hlo_ref[...], hhi_ref[...]], axis=1)
    y = jnp.maximum(h + m + b_ref[...], 0.0)
    ylo_ref[...] = y[:, :H]
    yhi_ref[...] = y[:, H:]

    @pl.when(i == 0)
    def _():
        ysum_ref[...] = jnp.zeros_like(ysum_ref)

    ysum_ref[...] += jnp.sum(y, axis=0, keepdims=True)


_update = pl.pallas_call(
    _update_body,
    grid=(GRID,),
    in_specs=[
        pl.BlockSpec((RBLK, H), lambda i: (i, 0)),
        pl.BlockSpec((RBLK, H), lambda i: (i, 0)),
        pl.BlockSpec((RBLK, H), lambda i: (i, 0)),
        pl.BlockSpec((RBLK, H), lambda i: (i, 0)),
        pl.BlockSpec((D, D), lambda i: (0, 0)),
        pl.BlockSpec((1, D), lambda i: (0, 0)),
    ],
    out_specs=[
        pl.BlockSpec((RBLK, H), lambda i: (i, 0)),
        pl.BlockSpec((RBLK, H), lambda i: (i, 0)),
        pl.BlockSpec((1, D), lambda i: (0, 0)),
    ],
    out_shape=[
        jax.ShapeDtypeStruct((N_NODES, H), jnp.float32),
        jax.ShapeDtypeStruct((N_NODES, H), jnp.float32),
        jax.ShapeDtypeStruct((1, D), jnp.float32),
    ],
)


def kernel(initial_state, edge_species, edge_reactions, propensity_type_ids,
           propensity_params, W_sp, b_sp, type_table, W_pp, b_pp,
           Ws2r, Wr2s, br, bs):
    n_layers = Ws2r.shape[0]

    es = edge_species.astype(jnp.int32)
    er = edge_reactions.astype(jnp.int32)
    pad = EPAD - E
    es_src = jnp.concatenate([es, jnp.zeros((pad,), jnp.int32)]).reshape(NT, TPC, K)
    er_src = jnp.concatenate([er, jnp.zeros((pad,), jnp.int32)]).reshape(NT, TPC, K)
    es_dst = jnp.concatenate([es, jnp.full((pad,), N_NODES, jnp.int32)]).reshape(NT, TPC, K)
    er_dst = jnp.concatenate([er, jnp.full((pad,), N_NODES, jnp.int32)]).reshape(NT, TPC, K)
    zeros = jnp.zeros((ACC_ROWS, H), jnp.float32)

    hs_lo, hs_hi, hr_lo, hr_hi = _embed(
        initial_state.reshape(N_NODES, 1), W_sp, b_sp.reshape(1, D), type_table,
        propensity_type_ids.astype(jnp.int32).reshape(N_NODES, 1),
        propensity_params, W_pp, b_pp.reshape(1, D))

    sum_r = sum_s = None
    for l in range(n_layers):
        a_lo, a_hi = _agg(hs_lo, hs_hi, es_src, er_dst, zeros)
        hr_lo, hr_hi, sum_r = _update(hr_lo, hr_hi, a_lo, a_hi,
                                      Ws2r[l], br[l].reshape(1, D))
        a_lo, a_hi = _agg(hr_lo, hr_hi, er_src, es_dst, zeros)
        hs_lo, hs_hi, sum_s = _update(hs_lo, hs_hi, a_lo, a_hi,
                                      Wr2s[l], bs[l].reshape(1, D))

    h_s = jnp.concatenate([hs_lo, hs_hi], axis=1)
    h_r = jnp.concatenate([hr_lo, hr_hi], axis=1)
    context = jnp.concatenate([sum_s[0], sum_r[0]]) * (1.0 / N_NODES)
    return h_s, h_r, context


# speedup vs baseline: 1.7956x; 1.0125x over previous
import jax
import jax.numpy as jnp
from jax import lax
from jax.experimental import pallas as pl
from jax.experimental.pallas import tpu as pltpu
from jax.experimental.pallas import tpu_sc as plsc

N_NODES = 10000
D = 256
H = 128
E = 160000
NT = 16
K = 48
TPC = 209
EPAD = NT * TPC * K
ACC_ROWS = 10112
STRIPE = ACC_ROWS // NT
TAIL = N_NODES - 15 * STRIPE
RBLK = 2000
GRID = N_NODES // RBLK
NBUF = 9
RB = 7
GA = 5
IA = 7
SL = 2


def _agg_body(h_lo, h_hi, src3, dst3, zeros_hbm, out_lo, out_hi,
              src_v, dst_v, rows_v, acc, isem, gsem, ssem, zsem):
    c = lax.axis_index("c")
    s = lax.axis_index("s")

    pltpu.async_copy(zeros_hbm.at[pl.ds(s * STRIPE, STRIPE)],
                     acc.at[pl.ds(s * STRIPE, STRIPE)], zsem)

    def idx_load(j, slot):
        pltpu.async_copy(src3.at[s].at[j], src_v.at[slot], isem.at[slot])
        pltpu.async_copy(dst3.at[s].at[j], dst_v.at[slot], isem.at[slot])

    def idx_wait(j, slot):
        pltpu.make_async_copy(src3.at[s].at[j], src_v.at[slot],
                              isem.at[slot]).wait()
        pltpu.make_async_copy(dst3.at[s].at[j], dst_v.at[slot],
                              isem.at[slot]).wait()

    def run(h_ref):
        for j in range(IA):
            idx_load(j, j)
        for j in range(GA):
            idx_wait(j, j)
            pltpu.async_copy(h_ref.at[src_v.at[j]], rows_v.at[j], gsem.at[j])

        pltpu.make_async_copy(zeros_hbm.at[pl.ds(s * STRIPE, STRIPE)],
                              acc.at[pl.ds(s * STRIPE, STRIPE)], zsem).wait()
        plsc.subcore_barrier()

        def chunk(j, carry):
            bi = lax.rem(j, NBUF)
            br = lax.rem(j, RB)
            bid = lax.rem(j + IA, NBUF)
            big = lax.rem(j + GA, NBUF)
            brg = lax.rem(j + GA, RB)
            pltpu.make_async_copy(h_ref.at[src_v.at[bi]], rows_v.at[br],
                                  gsem.at[br]).wait()
            pltpu.async_copy(rows_v.at[br], acc.at[dst_v.at[bi]], ssem.at[br],
                             add=True)

            @pl.when(j >= SL)
            def _():
                pltpu.make_async_copy(rows_v.at[brg], acc.at[dst_v.at[bid]],
                                      ssem.at[brg]).wait()

            @pl.when(j + IA < TPC)
            def _():
                idx_load(j + IA, bid)

            @pl.when(j + GA < TPC)
            def _():
                idx_wait(j + GA, big)
                pltpu.async_copy(h_ref.at[src_v.at[big]], rows_v.at[brg],
                                 gsem.at[brg])

            return carry
        lax.fori_loop(0, TPC, chunk, 0)

        for j in range(TPC - SL, TPC):
            pltpu.make_async_copy(rows_v.at[j % RB], acc.at[dst_v.at[j % NBUF]],
                                  ssem.at[j % RB]).wait()

    @pl.when(c == 0)
    def _():
        run(h_lo)

    @pl.when(c == 1)
    def _():
        run(h_hi)

    plsc.subcore_barrier()

    def writeback(out_ref):
        @pl.when(s < NT - 1)
        def _():
            pltpu.sync_copy(acc.at[pl.ds(s * STRIPE, STRIPE)],
                            out_ref.at[pl.ds(s * STRIPE, STRIPE)])

        @pl.when(s == NT - 1)
        def _():
            pltpu.sync_copy(acc.at[pl.ds((NT - 1) * STRIPE, TAIL)],
                            out_ref.at[pl.ds((NT - 1) * STRIPE, TAIL)])

    @pl.when(c == 0)
    def _():
        writeback(out_lo)

    @pl.when(c == 1)
    def _():
        writeback(out_hi)


_agg_cache = []


def _agg(*args):
    if not _agg_cache:
        _agg_cache.append(pl.kernel(
            _agg_body,
            out_type=(jax.ShapeDtypeStruct((N_NODES, H), jnp.float32),
                      jax.ShapeDtypeStruct((N_NODES, H), jnp.float32)),
            mesh=plsc.VectorSubcoreMesh(core_axis_name="c", subcore_axis_name="s",
                                        num_cores=2, num_subcores=NT),
            scratch_types=[
                pltpu.VMEM((NBUF, K), jnp.int32),
                pltpu.VMEM((NBUF, K), jnp.int32),
                pltpu.VMEM((RB, K, H), jnp.float32),
                pltpu.VMEM_SHARED((ACC_ROWS, H), jnp.float32),
                pltpu.SemaphoreType.DMA((NBUF,)),
                pltpu.SemaphoreType.DMA((RB,)),
                pltpu.SemaphoreType.DMA((RB,)),
                pltpu.SemaphoreType.DMA,
            ],
        ))
    return _agg_cache[0](*args)


def _embed_body(x_ref, wsp_ref, bsp_ref, tt_ref, pt_ref, pp_ref, wpp_ref,
                bpp_ref, hs_lo, hs_hi, hr_lo, hr_hi):
    feat = jnp.log1p(x_ref[...])
    hs = jnp.tanh(feat * wsp_ref[...] + bsp_ref[...])
    hs_lo[...] = hs[:, :H]
    hs_hi[...] = hs[:, H:]

    ids = pt_ref[...]
    oh = (ids == lax.broadcasted_iota(jnp.int32, (ids.shape[0], 8), 1))
    emb = jnp.dot(oh.astype(jnp.float32), tt_ref[...],
                  preferred_element_type=jnp.float32)
    pp = pp_ref[...]
    lin = jnp.zeros_like(emb)
    for kk in range(4):
        lin = lin + pp[:, kk:kk + 1] * wpp_ref[kk:kk + 1, :]
    hr = jnp.tanh(emb + lin + bpp_ref[...])
    hr_lo[...] = hr[:, :H]
    hr_hi[...] = hr[:, H:]


_embed = pl.pallas_call(
    _embed_body,
    grid=(GRID,),
    in_specs=[
        pl.BlockSpec((RBLK, 1), lambda i: (i, 0)),
        pl.BlockSpec((1, D), lambda i: (0, 0)),
        pl.BlockSpec((1, D), lambda i: (0, 0)),
        pl.BlockSpec((8, D), lambda i: (0, 0)),
        pl.BlockSpec((RBLK, 1), lambda i: (i, 0)),
        pl.BlockSpec((RBLK, 4), lambda i: (i, 0)),
        pl.BlockSpec((4, D), lambda i: (0, 0)),
        pl.BlockSpec((1, D), lambda i: (0, 0)),
    ],
    out_specs=[
        pl.BlockSpec((RBLK, H), lambda i: (i, 0)),
        pl.BlockSpec((RBLK, H), lambda i: (i, 0)),
        pl.BlockSpec((RBLK, H), lambda i: (i, 0)),
        pl.BlockSpec((RBLK, H), lambda i: (i, 0)),
    ],
    out_shape=[jax.ShapeDtypeStruct((N_NODES, H), jnp.float32)] * 4,
)


def _update_body(hlo_ref, hhi_ref, alo_ref, ahi_ref, w_ref, b_ref,
                 ylo_ref, yhi_ref, ysum_ref):
    i = pl.program_id(0)
    m = (jnp.dot(alo_ref[...], w_ref[:H, :], preferred_element_type=jnp.float32)
         + jnp.dot(ahi_ref[...], w_ref[H:, :], preferred_element_type=jnp.float32))
    h = jnp.concatenate([hlo_ref[...], hhi_ref[...]], axis=1)
    y = jnp.maximum(h + m + b_ref[...], 0.0)
    ylo_ref[...] = y[:, :H]
    yhi_ref[...] = y[:, H:]

    @pl.when(i == 0)
    def _():
        ysum_ref[...] = jnp.zeros_like(ysum_ref)

    ysum_ref[...] += jnp.sum(y, axis=0, keepdims=True)


_update = pl.pallas_call(
    _update_body,
    grid=(GRID,),
    in_specs=[
        pl.BlockSpec((RBLK, H), lambda i: (i, 0)),
        pl.BlockSpec((RBLK, H), lambda i: (i, 0)),
        pl.BlockSpec((RBLK, H), lambda i: (i, 0)),
        pl.BlockSpec((RBLK, H), lambda i: (i, 0)),
        pl.BlockSpec((D, D), lambda i: (0, 0)),
        pl.BlockSpec((1, D), lambda i: (0, 0)),
    ],
    out_specs=[
        pl.BlockSpec((RBLK, H), lambda i: (i, 0)),
        pl.BlockSpec((RBLK, H), lambda i: (i, 0)),
        pl.BlockSpec((1, D), lambda i: (0, 0)),
    ],
    out_shape=[
        jax.ShapeDtypeStruct((N_NODES, H), jnp.float32),
        jax.ShapeDtypeStruct((N_NODES, H), jnp.float32),
        jax.ShapeDtypeStruct((1, D), jnp.float32),
    ],
)


def kernel(initial_state, edge_species, edge_reactions, propensity_type_ids,
           propensity_params, W_sp, b_sp, type_table, W_pp, b_pp,
           Ws2r, Wr2s, br, bs):
    n_layers = Ws2r.shape[0]

    es = edge_species.astype(jnp.int32)
    er = edge_reactions.astype(jnp.int32)
    pad = EPAD - E
    es_src = jnp.concatenate([es, jnp.zeros((pad,), jnp.int32)]).reshape(NT, TPC, K)
    er_src = jnp.concatenate([er, jnp.zeros((pad,), jnp.int32)]).reshape(NT, TPC, K)
    es_dst = jnp.concatenate([es, jnp.full((pad,), N_NODES, jnp.int32)]).reshape(NT, TPC, K)
    er_dst = jnp.concatenate([er, jnp.full((pad,), N_NODES, jnp.int32)]).reshape(NT, TPC, K)
    zeros = jnp.zeros((ACC_ROWS, H), jnp.float32)

    hs_lo, hs_hi, hr_lo, hr_hi = _embed(
        initial_state.reshape(N_NODES, 1), W_sp, b_sp.reshape(1, D), type_table,
        propensity_type_ids.astype(jnp.int32).reshape(N_NODES, 1),
        propensity_params, W_pp, b_pp.reshape(1, D))

    sum_r = sum_s = None
    for l in range(n_layers):
        a_lo, a_hi = _agg(hs_lo, hs_hi, es_src, er_dst, zeros)
        hr_lo, hr_hi, sum_r = _update(hr_lo, hr_hi, a_lo, a_hi,
                                      Ws2r[l], br[l].reshape(1, D))
        a_lo, a_hi = _agg(hr_lo, hr_hi, er_src, es_dst, zeros)
        hs_lo, hs_hi, sum_s = _update(hs_lo, hs_hi, a_lo, a_hi,
                                      Wr2s[l], bs[l].reshape(1, D))

    h_s = jnp.concatenate([hs_lo, hs_hi], axis=1)
    h_r = jnp.concatenate([hr_lo, hr_hi], axis=1)
    context = jnp.concatenate([sum_s[0], sum_r[0]]) * (1.0 / N_NODES)
    return h_s, h_r, context


# speedup vs baseline: 2.1799x; 1.2140x over previous
import jax
import jax.numpy as jnp
from jax import lax
from jax.experimental import pallas as pl
from jax.experimental.pallas import tpu as pltpu
from jax.experimental.pallas import tpu_sc as plsc

N_NODES = 10000
D = 256
H = 128
E = 160000
NT = 16
K = 40
TPC = 250
EPAD = NT * TPC * K
ACC_ROWS = 10112
STRIPE = ACC_ROWS // NT
TAIL = N_NODES - 15 * STRIPE
RBLK = 2000
GRID = N_NODES // RBLK
NBUF = 10
RB = 8
GA = 6
IA = 8
SL = 2


def _agg_body(h_lo, h_hi, src3, dst3, zeros_hbm, out_lo, out_hi,
              src_v, dst_v, rows_v, acc, isem, gsem, ssem, zsem):
    c = lax.axis_index("c")
    s = lax.axis_index("s")

    pltpu.async_copy(zeros_hbm.at[pl.ds(s * STRIPE, STRIPE)],
                     acc.at[pl.ds(s * STRIPE, STRIPE)], zsem)

    def idx_load(j, slot):
        pltpu.async_copy(src3.at[s].at[j], src_v.at[slot], isem.at[slot])
        pltpu.async_copy(dst3.at[s].at[j], dst_v.at[slot], isem.at[slot])

    def idx_wait(j, slot):
        pltpu.make_async_copy(src3.at[s].at[j], src_v.at[slot],
                              isem.at[slot]).wait()
        pltpu.make_async_copy(dst3.at[s].at[j], dst_v.at[slot],
                              isem.at[slot]).wait()

    def run(h_ref):
        for j in range(IA):
            idx_load(j, j)
        for j in range(GA):
            idx_wait(j, j)
            pltpu.async_copy(h_ref.at[src_v.at[j]], rows_v.at[j], gsem.at[j])

        pltpu.make_async_copy(zeros_hbm.at[pl.ds(s * STRIPE, STRIPE)],
                              acc.at[pl.ds(s * STRIPE, STRIPE)], zsem).wait()
        plsc.subcore_barrier()

        def chunk(j, carry):
            bi = lax.rem(j, NBUF)
            br = lax.rem(j, RB)
            bid = lax.rem(j + IA, NBUF)
            big = lax.rem(j + GA, NBUF)
            brg = lax.rem(j + GA, RB)
            pltpu.make_async_copy(h_ref.at[src_v.at[bi]], rows_v.at[br],
                                  gsem.at[br]).wait()
            pltpu.async_copy(rows_v.at[br], acc.at[dst_v.at[bi]], ssem.at[br],
                             add=True)

            @pl.when(j >= SL)
            def _():
                pltpu.make_async_copy(rows_v.at[brg], acc.at[dst_v.at[bid]],
                                      ssem.at[brg]).wait()

            @pl.when(j + IA < TPC)
            def _():
                idx_load(j + IA, bid)

            @pl.when(j + GA < TPC)
            def _():
                idx_wait(j + GA, big)
                pltpu.async_copy(h_ref.at[src_v.at[big]], rows_v.at[brg],
                                 gsem.at[brg])

            return carry
        lax.fori_loop(0, TPC, chunk, 0)

        for j in range(TPC - SL, TPC):
            pltpu.make_async_copy(rows_v.at[j % RB], acc.at[dst_v.at[j % NBUF]],
                                  ssem.at[j % RB]).wait()

    @pl.when(c == 0)
    def _():
        run(h_lo)

    @pl.when(c == 1)
    def _():
        run(h_hi)

    plsc.subcore_barrier()

    def writeback(out_ref):
        @pl.when(s < NT - 1)
        def _():
            pltpu.sync_copy(acc.at[pl.ds(s * STRIPE, STRIPE)],
                            out_ref.at[pl.ds(s * STRIPE, STRIPE)])

        @pl.when(s == NT - 1)
        def _():
            pltpu.sync_copy(acc.at[pl.ds((NT - 1) * STRIPE, TAIL)],
                            out_ref.at[pl.ds((NT - 1) * STRIPE, TAIL)])

    @pl.when(c == 0)
    def _():
        writeback(out_lo)

    @pl.when(c == 1)
    def _():
        writeback(out_hi)


_agg_cache = []


def _agg(*args):
    if not _agg_cache:
        _agg_cache.append(pl.kernel(
            _agg_body,
            out_type=(jax.ShapeDtypeStruct((N_NODES, H), jnp.float32),
                      jax.ShapeDtypeStruct((N_NODES, H), jnp.float32)),
            mesh=plsc.VectorSubcoreMesh(core_axis_name="c", subcore_axis_name="s",
                                        num_cores=2, num_subcores=NT),
            scratch_types=[
                pltpu.VMEM((NBUF, K), jnp.int32),
                pltpu.VMEM((NBUF, K), jnp.int32),
                pltpu.VMEM((RB, K, H), jnp.float32),
                pltpu.VMEM_SHARED((ACC_ROWS, H), jnp.float32),
                pltpu.SemaphoreType.DMA((NBUF,)),
                pltpu.SemaphoreType.DMA((RB,)),
                pltpu.SemaphoreType.DMA((RB,)),
                pltpu.SemaphoreType.DMA,
            ],
        ))
    return _agg_cache[0](*args)


def _embed_body(x_ref, wsp_ref, bsp_ref, tt_ref, pt_ref, pp_ref, wpp_ref,
                bpp_ref, hs_lo, hs_hi, hr_lo, hr_hi):
    feat = jnp.log1p(x_ref[...])
    hs = jnp.tanh(feat * wsp_ref[...] + bsp_ref[...])
    hs_lo[...] = hs[:, :H]
    hs_hi[...] = hs[:, H:]

    ids = pt_ref[...]
    oh = (ids == lax.broadcasted_iota(jnp.int32, (ids.shape[0], 8), 1))
    emb = jnp.dot(oh.astype(jnp.float32), tt_ref[...],
                  preferred_element_type=jnp.float32)
    pp = pp_ref[...]
    lin = jnp.zeros_like(emb)
    for kk in range(4):
        lin = lin + pp[:, kk:kk + 1] * wpp_ref[kk:kk + 1, :]
    hr = jnp.tanh(emb + lin + bpp_ref[...])
    hr_lo[...] = hr[:, :H]
    hr_hi[...] = hr[:, H:]


_embed = pl.pallas_call(
    _embed_body,
    grid=(GRID,),
    in_specs=[
        pl.BlockSpec((RBLK, 1), lambda i: (i, 0)),
        pl.BlockSpec((1, D), lambda i: (0, 0)),
        pl.BlockSpec((1, D), lambda i: (0, 0)),
        pl.BlockSpec((8, D), lambda i: (0, 0)),
        pl.BlockSpec((RBLK, 1), lambda i: (i, 0)),
        pl.BlockSpec((RBLK, 4), lambda i: (i, 0)),
        pl.BlockSpec((4, D), lambda i: (0, 0)),
        pl.BlockSpec((1, D), lambda i: (0, 0)),
    ],
    out_specs=[
        pl.BlockSpec((RBLK, H), lambda i: (i, 0)),
        pl.BlockSpec((RBLK, H), lambda i: (i, 0)),
        pl.BlockSpec((RBLK, H), lambda i: (i, 0)),
        pl.BlockSpec((RBLK, H), lambda i: (i, 0)),
    ],
    out_shape=[jax.ShapeDtypeStruct((N_NODES, H), jnp.float32)] * 4,
)


def _update_body(hlo_ref, hhi_ref, alo_ref, ahi_ref, w_ref, b_ref,
                 ylo_ref, yhi_ref, ysum_ref):
    i = pl.program_id(0)
    m = (jnp.dot(alo_ref[...], w_ref[:H, :], preferred_element_type=jnp.float32)
         + jnp.dot(ahi_ref[...], w_ref[H:, :], preferred_element_type=jnp.float32))
    h = jnp.concatenate([hlo_ref[...], hhi_ref[...]], axis=1)
    y = jnp.maximum(h + m + b_ref[...], 0.0)
    ylo_ref[...] = y[:, :H]
    yhi_ref[...] = y[:, H:]

    @pl.when(i == 0)
    def _():
        ysum_ref[...] = jnp.zeros_like(ysum_ref)

    ysum_ref[...] += jnp.sum(y, axis=0, keepdims=True)


_update = pl.pallas_call(
    _update_body,
    grid=(GRID,),
    in_specs=[
        pl.BlockSpec((RBLK, H), lambda i: (i, 0)),
        pl.BlockSpec((RBLK, H), lambda i: (i, 0)),
        pl.BlockSpec((RBLK, H), lambda i: (i, 0)),
        pl.BlockSpec((RBLK, H), lambda i: (i, 0)),
        pl.BlockSpec((D, D), lambda i: (0, 0)),
        pl.BlockSpec((1, D), lambda i: (0, 0)),
    ],
    out_specs=[
        pl.BlockSpec((RBLK, H), lambda i: (i, 0)),
        pl.BlockSpec((RBLK, H), lambda i: (i, 0)),
        pl.BlockSpec((1, D), lambda i: (0, 0)),
    ],
    out_shape=[
        jax.ShapeDtypeStruct((N_NODES, H), jnp.float32),
        jax.ShapeDtypeStruct((N_NODES, H), jnp.float32),
        jax.ShapeDtypeStruct((1, D), jnp.float32),
    ],
)


def kernel(initial_state, edge_species, edge_reactions, propensity_type_ids,
           propensity_params, W_sp, b_sp, type_table, W_pp, b_pp,
           Ws2r, Wr2s, br, bs):
    n_layers = Ws2r.shape[0]

    es = edge_species.astype(jnp.int32)
    er = edge_reactions.astype(jnp.int32)
    pad = EPAD - E
    es_src = jnp.concatenate([es, jnp.zeros((pad,), jnp.int32)]).reshape(NT, TPC, K)
    er_src = jnp.concatenate([er, jnp.zeros((pad,), jnp.int32)]).reshape(NT, TPC, K)
    es_dst = jnp.concatenate([es, jnp.full((pad,), N_NODES, jnp.int32)]).reshape(NT, TPC, K)
    er_dst = jnp.concatenate([er, jnp.full((pad,), N_NODES, jnp.int32)]).reshape(NT, TPC, K)
    zeros = jnp.zeros((ACC_ROWS, H), jnp.float32)

    hs_lo, hs_hi, hr_lo, hr_hi = _embed(
        initial_state.reshape(N_NODES, 1), W_sp, b_sp.reshape(1, D), type_table,
        propensity_type_ids.astype(jnp.int32).reshape(N_NODES, 1),
        propensity_params, W_pp, b_pp.reshape(1, D))

    sum_r = sum_s = None
    for l in range(n_layers):
        a_lo, a_hi = _agg(hs_lo, hs_hi, es_src, er_dst, zeros)
        hr_lo, hr_hi, sum_r = _update(hr_lo, hr_hi, a_lo, a_hi,
                                      Ws2r[l], br[l].reshape(1, D))
        a_lo, a_hi = _agg(hr_lo, hr_hi, er_src, es_dst, zeros)
        hs_lo, hs_hi, sum_s = _update(hs_lo, hs_hi, a_lo, a_hi,
                                      Wr2s[l], bs[l].reshape(1, D))

    h_s = jnp.concatenate([hs_lo, hs_hi], axis=1)
    h_r = jnp.concatenate([hr_lo, hr_hi], axis=1)
    context = jnp.concatenate([sum_s[0], sum_r[0]]) * (1.0 / N_NODES)
    return h_s, h_r, context


# speedup vs baseline: 2.1866x; 1.0031x over previous
import jax
import jax.numpy as jnp
from jax import lax
from jax.experimental import pallas as pl
from jax.experimental.pallas import tpu as pltpu
from jax.experimental.pallas import tpu_sc as plsc

N_NODES = 10000
D = 256
H = 128
E = 160000
NT = 16
K = 40
TPC = 250
EPAD = NT * TPC * K
ACC_ROWS = 10112
STRIPE = ACC_ROWS // NT
TAIL = N_NODES - 15 * STRIPE
RBLK = 2000
GRID = N_NODES // RBLK
NBUF = 11
RB = 9
GA = 7
IA = 9
SL = 2


def _agg_body(h_lo, h_hi, src3, dst3, zeros_hbm, out_lo, out_hi,
              src_v, dst_v, rows_v, acc, isem, gsem, ssem, zsem):
    c = lax.axis_index("c")
    s = lax.axis_index("s")

    pltpu.async_copy(zeros_hbm.at[pl.ds(s * STRIPE, STRIPE)],
                     acc.at[pl.ds(s * STRIPE, STRIPE)], zsem)

    def idx_load(j, slot):
        pltpu.async_copy(src3.at[s].at[j], src_v.at[slot], isem.at[slot])
        pltpu.async_copy(dst3.at[s].at[j], dst_v.at[slot], isem.at[slot])

    def idx_wait(j, slot):
        pltpu.make_async_copy(src3.at[s].at[j], src_v.at[slot],
                              isem.at[slot]).wait()
        pltpu.make_async_copy(dst3.at[s].at[j], dst_v.at[slot],
                              isem.at[slot]).wait()

    def run(h_ref):
        for j in range(IA):
            idx_load(j, j)
        for j in range(GA):
            idx_wait(j, j)
            pltpu.async_copy(h_ref.at[src_v.at[j]], rows_v.at[j], gsem.at[j])

        pltpu.make_async_copy(zeros_hbm.at[pl.ds(s * STRIPE, STRIPE)],
                              acc.at[pl.ds(s * STRIPE, STRIPE)], zsem).wait()
        plsc.subcore_barrier()

        def chunk(j, carry):
            bi = lax.rem(j, NBUF)
            br = lax.rem(j, RB)
            bid = lax.rem(j + IA, NBUF)
            big = lax.rem(j + GA, NBUF)
            brg = lax.rem(j + GA, RB)
            pltpu.make_async_copy(h_ref.at[src_v.at[bi]], rows_v.at[br],
                                  gsem.at[br]).wait()
            pltpu.async_copy(rows_v.at[br], acc.at[dst_v.at[bi]], ssem.at[br],
                             add=True)

            @pl.when(j >= SL)
            def _():
                pltpu.make_async_copy(rows_v.at[brg], acc.at[dst_v.at[bid]],
                                      ssem.at[brg]).wait()

            @pl.when(j + IA < TPC)
            def _():
                idx_load(j + IA, bid)

            @pl.when(j + GA < TPC)
            def _():
                idx_wait(j + GA, big)
                pltpu.async_copy(h_ref.at[src_v.at[big]], rows_v.at[brg],
                                 gsem.at[brg])

            return carry
        lax.fori_loop(0, TPC, chunk, 0)

        for j in range(TPC - SL, TPC):
            pltpu.make_async_copy(rows_v.at[j % RB], acc.at[dst_v.at[j % NBUF]],
                                  ssem.at[j % RB]).wait()

    @pl.when(c == 0)
    def _():
        run(h_lo)

    @pl.when(c == 1)
    def _():
        run(h_hi)

    plsc.subcore_barrier()

    def writeback(out_ref):
        @pl.when(s < NT - 1)
        def _():
            pltpu.sync_copy(acc.at[pl.ds(s * STRIPE, STRIPE)],
                            out_ref.at[pl.ds(s * STRIPE, STRIPE)])

        @pl.when(s == NT - 1)
        def _():
            pltpu.sync_copy(acc.at[pl.ds((NT - 1) * STRIPE, TAIL)],
                            out_ref.at[pl.ds((NT - 1) * STRIPE, TAIL)])

    @pl.when(c == 0)
    def _():
        writeback(out_lo)

    @pl.when(c == 1)
    def _():
        writeback(out_hi)


_agg_cache = []


def _agg(*args):
    if not _agg_cache:
        _agg_cache.append(pl.kernel(
            _agg_body,
            out_type=(jax.ShapeDtypeStruct((N_NODES, H), jnp.float32),
                      jax.ShapeDtypeStruct((N_NODES, H), jnp.float32)),
            mesh=plsc.VectorSubcoreMesh(core_axis_name="c", subcore_axis_name="s",
                                        num_cores=2, num_subcores=NT),
            scratch_types=[
                pltpu.VMEM((NBUF, K), jnp.int32),
                pltpu.VMEM((NBUF, K), jnp.int32),
                pltpu.VMEM((RB, K, H), jnp.float32),
                pltpu.VMEM_SHARED((ACC_ROWS, H), jnp.float32),
                pltpu.SemaphoreType.DMA((NBUF,)),
                pltpu.SemaphoreType.DMA((RB,)),
                pltpu.SemaphoreType.DMA((RB,)),
                pltpu.SemaphoreType.DMA,
            ],
        ))
    return _agg_cache[0](*args)


def _embed_body(x_ref, wsp_ref, bsp_ref, tt_ref, pt_ref, pp_ref, wpp_ref,
                bpp_ref, hs_lo, hs_hi, hr_lo, hr_hi):
    feat = jnp.log1p(x_ref[...])
    hs = jnp.tanh(feat * wsp_ref[...] + bsp_ref[...])
    hs_lo[...] = hs[:, :H]
    hs_hi[...] = hs[:, H:]

    ids = pt_ref[...]
    oh = (ids == lax.broadcasted_iota(jnp.int32, (ids.shape[0], 8), 1))
    emb = jnp.dot(oh.astype(jnp.float32), tt_ref[...],
                  preferred_element_type=jnp.float32)
    pp = pp_ref[...]
    lin = jnp.zeros_like(emb)
    for kk in range(4):
        lin = lin + pp[:, kk:kk + 1] * wpp_ref[kk:kk + 1, :]
    hr = jnp.tanh(emb + lin + bpp_ref[...])
    hr_lo[...] = hr[:, :H]
    hr_hi[...] = hr[:, H:]


_embed = pl.pallas_call(
    _embed_body,
    grid=(GRID,),
    in_specs=[
        pl.BlockSpec((RBLK, 1), lambda i: (i, 0)),
        pl.BlockSpec((1, D), lambda i: (0, 0)),
        pl.BlockSpec((1, D), lambda i: (0, 0)),
        pl.BlockSpec((8, D), lambda i: (0, 0)),
        pl.BlockSpec((RBLK, 1), lambda i: (i, 0)),
        pl.BlockSpec((RBLK, 4), lambda i: (i, 0)),
        pl.BlockSpec((4, D), lambda i: (0, 0)),
        pl.BlockSpec((1, D), lambda i: (0, 0)),
    ],
    out_specs=[
        pl.BlockSpec((RBLK, H), lambda i: (i, 0)),
        pl.BlockSpec((RBLK, H), lambda i: (i, 0)),
        pl.BlockSpec((RBLK, H), lambda i: (i, 0)),
        pl.BlockSpec((RBLK, H), lambda i: (i, 0)),
    ],
    out_shape=[jax.ShapeDtypeStruct((N_NODES, H), jnp.float32)] * 4,
)


def _update_body(hlo_ref, hhi_ref, alo_ref, ahi_ref, w_ref, b_ref,
                 ylo_ref, yhi_ref, ysum_ref):
    i = pl.program_id(0)
    m = (jnp.dot(alo_ref[...], w_ref[:H, :], preferred_element_type=jnp.float32)
         + jnp.dot(ahi_ref[...], w_ref[H:, :], preferred_element_type=jnp.float32))
    h = jnp.concatenate([hlo_ref[...], hhi_ref[...]], axis=1)
    y = jnp.maximum(h + m + b_ref[...], 0.0)
    ylo_ref[...] = y[:, :H]
    yhi_ref[...] = y[:, H:]

    @pl.when(i == 0)
    def _():
        ysum_ref[...] = jnp.zeros_like(ysum_ref)

    ysum_ref[...] += jnp.sum(y, axis=0, keepdims=True)


_update = pl.pallas_call(
    _update_body,
    grid=(GRID,),
    in_specs=[
        pl.BlockSpec((RBLK, H), lambda i: (i, 0)),
        pl.BlockSpec((RBLK, H), lambda i: (i, 0)),
        pl.BlockSpec((RBLK, H), lambda i: (i, 0)),
        pl.BlockSpec((RBLK, H), lambda i: (i, 0)),
        pl.BlockSpec((D, D), lambda i: (0, 0)),
        pl.BlockSpec((1, D), lambda i: (0, 0)),
    ],
    out_specs=[
        pl.BlockSpec((RBLK, H), lambda i: (i, 0)),
        pl.BlockSpec((RBLK, H), lambda i: (i, 0)),
        pl.BlockSpec((1, D), lambda i: (0, 0)),
    ],
    out_shape=[
        jax.ShapeDtypeStruct((N_NODES, H), jnp.float32),
        jax.ShapeDtypeStruct((N_NODES, H), jnp.float32),
        jax.ShapeDtypeStruct((1, D), jnp.float32),
    ],
)


def kernel(initial_state, edge_species, edge_reactions, propensity_type_ids,
           propensity_params, W_sp, b_sp, type_table, W_pp, b_pp,
           Ws2r, Wr2s, br, bs):
    n_layers = Ws2r.shape[0]

    es = edge_species.astype(jnp.int32)
    er = edge_reactions.astype(jnp.int32)
    pad = EPAD - E
    es_src = jnp.concatenate([es, jnp.zeros((pad,), jnp.int32)]).reshape(NT, TPC, K)
    er_src = jnp.concatenate([er, jnp.zeros((pad,), jnp.int32)]).reshape(NT, TPC, K)
    es_dst = jnp.concatenate([es, jnp.full((pad,), N_NODES, jnp.int32)]).reshape(NT, TPC, K)
    er_dst = jnp.concatenate([er, jnp.full((pad,), N_NODES, jnp.int32)]).reshape(NT, TPC, K)
    zeros = jnp.zeros((ACC_ROWS, H), jnp.float32)

    hs_lo, hs_hi, hr_lo, hr_hi = _embed(
        initial_state.reshape(N_NODES, 1), W_sp, b_sp.reshape(1, D), type_table,
        propensity_type_ids.astype(jnp.int32).reshape(N_NODES, 1),
        propensity_params, W_pp, b_pp.reshape(1, D))

    sum_r = sum_s = None
    for l in range(n_layers):
        a_lo, a_hi = _agg(hs_lo, hs_hi, es_src, er_dst, zeros)
        hr_lo, hr_hi, sum_r = _update(hr_lo, hr_hi, a_lo, a_hi,
                                      Ws2r[l], br[l].reshape(1, D))
        a_lo, a_hi = _agg(hr_lo, hr_hi, er_src, es_dst, zeros)
        hs_lo, hs_hi, sum_s = _update(hs_lo, hs_hi, a_lo, a_hi,
                                      Wr2s[l], bs[l].reshape(1, D))

    h_s = jnp.concatenate([hs_lo, hs_hi], axis=1)
    h_r = jnp.concatenate([hr_lo, hr_hi], axis=1)
    context = jnp.concatenate([sum_s[0], sum_r[0]]) * (1.0 / N_NODES)
    return h_s, h_r, context
